# baseline (speedup 1.0000x reference)
import dataclasses
import functools

import jax
import jax.numpy as jnp
from jax import lax
from jax.experimental import pallas as pl
from jax.experimental.pallas import tpu as pltpu
from jax.experimental.pallas import tpu_sc as plsc

_NUM_WORKERS = 32
_CHUNK = 2048
_GSUB = 128


def _sc_grid_sample(tab0, tab1, grid_lin, *, H, W, N, Ho, Wo):
    HOWO = Ho * Wo
    HW = H * W
    imgs_per_worker = N // _NUM_WORKERS
    chunks_per_img = HOWO // _CHUNK
    nch = imgs_per_worker * chunks_per_img
    sx = (W - 1) / 2.0
    sy = (H - 1) / 2.0

    mesh = plsc.VectorSubcoreMesh(core_axis_name="c", subcore_axis_name="s")

    cp = pltpu.CompilerParams()
    for fld, val in (("needs_layout_passes", False),
                     ("use_tc_tiling_on_sc", False)):
        if fld in pltpu.CompilerParams.__dataclass_fields__:
            cp = dataclasses.replace(cp, **{fld: val})

    vm = pltpu.VMEM
    scratch = []
    for _ in range(2):
        scratch += [
            vm((2 * _CHUNK,), jnp.float32),
            vm((_CHUNK,), jnp.int32),
            vm((_CHUNK,), jnp.int32),
            vm((_CHUNK,), jnp.float32),
            vm((_CHUNK,), jnp.float32),
            vm((_CHUNK,), jnp.int32),
            vm((_CHUNK,), jnp.int32),
            vm((_CHUNK,), jnp.int32),
            vm((_CHUNK,), jnp.int32),
            vm((_CHUNK,), jnp.float32),
            vm((_CHUNK,), jnp.float32),
            pltpu.SemaphoreType.DMA,
            pltpu.SemaphoreType.DMA,
            pltpu.SemaphoreType.DMA,
        ]

    @functools.partial(
        pl.kernel,
        compiler_params=cp,
        out_type=jax.ShapeDtypeStruct((N * 2 * HOWO,), jnp.float32),
        mesh=mesh,
        scratch_types=scratch,
    )
    def sc_kernel(tab0_hbm, tab1_hbm, grid_hbm, out_hbm, *bufs):
        cid = lax.axis_index("c")
        sid = lax.axis_index("s")
        wid = sid * 2 + cid

        names = ("g", "idx", "idxr", "wx", "wy", "l0", "r0", "l1", "r1",
                 "o0", "o1", "gsem", "rsem", "osem")
        B = [dict(zip(names, bufs[:14])), dict(zip(names, bufs[14:]))]

        def p_off(t):
            n = wid * imgs_per_worker + t // chunks_per_img
            j = lax.rem(t, chunks_per_img)
            return n * HOWO + j * _CHUNK

        def o_off(t):
            n = wid * imgs_per_worker + t // chunks_per_img
            j = lax.rem(t, chunks_per_img)
            return n * 2 * HOWO + j * _CHUNK

        def start_grid(t, b):
            pltpu.async_copy(grid_hbm.at[pl.ds(2 * p_off(t), 2 * _CHUNK)],
                             B[b]["g"], B[b]["gsem"])

        def wait_grid(b):
            pltpu.make_async_copy(grid_hbm.at[pl.ds(0, 2 * _CHUNK)],
                                  B[b]["g"], B[b]["gsem"]).wait()

        def compute(b):
            g_v, idx_v, idxr_v = B[b]["g"], B[b]["idx"], B[b]["idxr"]
            wx_v, wy_v = B[b]["wx"], B[b]["wy"]

            @pl.loop(0, _CHUNK, step=32)
            def _(i):
                for u in (0, 16):
                    iu = i + u
                    base = (iu // 128) * 256 + lax.rem(iu, 128)
                    gx = g_v[pl.ds(base, 16)]
                    gy = g_v[pl.ds(base + 128, 16)]
                    x = jnp.minimum(gx * sx + sx, W - 1.0)
                    y = jnp.minimum(gy * sy + sy, H - 1.0)
                    xi = x.astype(jnp.int32)
                    yi = y.astype(jnp.int32)
                    idx = xi * H + yi
                    idx_v[pl.ds(iu, 16)] = idx
                    idxr_v[pl.ds(iu, 16)] = jnp.minimum(idx + H, HW - 1)
                    wx_v[pl.ds(iu, 16)] = x - xi.astype(jnp.float32)
                    wy_v[pl.ds(iu, 16)] = y - yi.astype(jnp.float32)

        def gather_copies(b):
            d = B[b]
            for k in range(_CHUNK // _GSUB):
                s = pl.ds(k * _GSUB, _GSUB)
                yield pltpu.make_async_copy(
                    tab0_hbm.at[d["idx"].at[s]], d["l0"].at[s], d["rsem"])
                yield pltpu.make_async_copy(
                    tab0_hbm.at[d["idxr"].at[s]], d["r0"].at[s], d["rsem"])
                yield pltpu.make_async_copy(
                    tab1_hbm.at[d["idx"].at[s]], d["l1"].at[s], d["rsem"])
                yield pltpu.make_async_copy(
                    tab1_hbm.at[d["idxr"].at[s]], d["r1"].at[s], d["rsem"])

        def fire(b):
            for c in gather_copies(b):
                c.start()

        def drain(b):
            for c in gather_copies(b):
                c.wait()

        himask = jnp.full((16,), -65536, jnp.int32)

        def combine(b):
            d = B[b]
            l0_v, r0_v, l1_v, r1_v = d["l0"], d["r0"], d["l1"], d["r1"]
            wx_v, wy_v, out0_v, out1_v = d["wx"], d["wy"], d["o0"], d["o1"]

            @pl.loop(0, _CHUNK, step=32)
            def _(i):
                for u in (0, 16):
                    s = pl.ds(i + u, 16)
                    wx = wx_v[s]
                    wy = wy_v[s]
                    wl0 = l0_v[s]
                    wr0 = r0_v[s]
                    wl1 = l1_v[s]
                    wr1 = r1_v[s]
                    v00 = plsc.bitcast(wl0 << 16, jnp.float32)
                    v10 = plsc.bitcast(wl0 & himask, jnp.float32)
                    v01 = plsc.bitcast(wr0 << 16, jnp.float32)
                    v11 = plsc.bitcast(wr0 & himask, jnp.float32)
                    u00 = plsc.bitcast(wl1 << 16, jnp.float32)
                    u10 = plsc.bitcast(wl1 & himask, jnp.float32)
                    u01 = plsc.bitcast(wr1 << 16, jnp.float32)
                    u11 = plsc.bitcast(wr1 & himask, jnp.float32)
                    cl0 = v00 + wy * (v10 - v00)
                    cr0 = v01 + wy * (v11 - v01)
                    cl1 = u00 + wy * (u10 - u00)
                    cr1 = u01 + wy * (u11 - u01)
                    out0_v[s] = cl0 + wx * (cr0 - cl0)
                    out1_v[s] = cl1 + wx * (cr1 - cl1)

        def out_copies(t, b):
            d = B[b]
            o0 = o_off(t)
            yield pltpu.make_async_copy(
                d["o0"], out_hbm.at[pl.ds(o0, _CHUNK)], d["osem"])
            yield pltpu.make_async_copy(
                d["o1"], out_hbm.at[pl.ds(o0 + HOWO, _CHUNK)], d["osem"])

        def drain_out(b):
            for c in out_copies(0, b):
                c.wait()

        def stage_a(t, b):
            wait_grid(b)
            compute(b)
            fire(b)

        def stage_b(t, b):
            drain(b)

            @pl.when(t >= 2)
            def _():
                drain_out(b)

            combine(b)
            for c in out_copies(t, b):
                c.start()

        start_grid(0, 0)
        stage_a(0, 0)
        start_grid(1, 1)

        @pl.loop(0, nch, step=2)
        def _(t):
            stage_a(t + 1, 1)

            @pl.when(t + 2 < nch)
            def _():
                start_grid(t + 2, 0)

            stage_b(t, 0)

            @pl.when(t + 2 < nch)
            def _():
                stage_a(t + 2, 0)
                start_grid(t + 3, 1)

            stage_b(t + 1, 1)

        drain_out(0)
        drain_out(1)

    return sc_kernel(tab0, tab1, grid_lin)


def _pack_vert_pairs(plane_cm):
    lo = lax.bitcast_convert_type(
        plane_cm.astype(jnp.bfloat16), jnp.uint16).astype(jnp.uint32)
    hi = jnp.concatenate([lo[1:], lo[-1:]])
    return (lo | (hi << 16)).astype(jnp.int32)


def kernel(grid, inp):
    N, Ho, Wo, _ = grid.shape
    _, C, H, W = inp.shape
    assert C == 2 and N % _NUM_WORKERS == 0 and (Ho * Wo) % _CHUNK == 0
    assert Wo >= 128 and Wo % 128 == 0 and _CHUNK % 128 == 0

    tab0 = _pack_vert_pairs(inp[0, 0].T.reshape(-1))
    tab1 = _pack_vert_pairs(inp[0, 1].T.reshape(-1))
    grid_lin = grid.transpose(0, 1, 3, 2).reshape(-1)

    out_flat = _sc_grid_sample(tab0, tab1, grid_lin,
                               H=H, W=W, N=N, Ho=Ho, Wo=Wo)
    return out_flat.reshape(N, C, Ho, Wo)


# speedup vs baseline: 1.4701x; 1.4701x over previous
import dataclasses
import functools

import jax
import jax.numpy as jnp
from jax import lax
from jax.experimental import pallas as pl
from jax.experimental.pallas import tpu as pltpu
from jax.experimental.pallas import tpu_sc as plsc

_NUM_WORKERS = 32
_CHUNK = 2048
_GSUB = 128
_PBLK = 2000


def _sc_grid_sample(tab0, tab1, tab0s, tab1s, grid_lin, *, H, W, N, Ho, Wo):
    HOWO = Ho * Wo
    HW = H * W
    imgs_per_worker = N // _NUM_WORKERS
    chunks_per_img = HOWO // _CHUNK
    nch = imgs_per_worker * chunks_per_img
    nblk = HW // _PBLK
    blk_slots = -(-nblk // 16)
    if blk_slots % 2:
        blk_slots += 1
    sx = (W - 1) / 2.0
    sy = (H - 1) / 2.0

    mesh = plsc.VectorSubcoreMesh(core_axis_name="c", subcore_axis_name="s")

    cp = pltpu.CompilerParams()
    for fld, val in (("needs_layout_passes", False),
                     ("use_tc_tiling_on_sc", False)):
        if fld in pltpu.CompilerParams.__dataclass_fields__:
            cp = dataclasses.replace(cp, **{fld: val})

    vm = pltpu.VMEM
    scratch = []
    for _ in range(2):
        scratch += [
            vm((2 * _CHUNK,), jnp.float32),
            vm((_CHUNK,), jnp.int32),
            vm((_CHUNK,), jnp.float32),
            vm((_CHUNK,), jnp.float32),
            vm((_CHUNK, 8), jnp.int32),
            vm((_CHUNK,), jnp.float32),
            vm((_CHUNK,), jnp.float32),
            pltpu.SemaphoreType.DMA,
            pltpu.SemaphoreType.DMA,
            pltpu.SemaphoreType.DMA,
        ]
    scratch += [
        vm((_PBLK,), jnp.int32),
        vm((_PBLK,), jnp.int32),
        vm((_PBLK,), jnp.int32),
        vm((_PBLK,), jnp.int32),
        vm((_PBLK, 8), jnp.int32),
        vm((_PBLK, 8), jnp.int32),
        pltpu.SemaphoreType.DMA,
        pltpu.SemaphoreType.DMA,
    ]

    out_types = (
        jax.ShapeDtypeStruct((N * 2 * HOWO,), jnp.float32),
        jax.ShapeDtypeStruct((2 * HW, 8), jnp.int32),
    )

    @functools.partial(
        pl.kernel,
        compiler_params=cp,
        out_type=out_types,
        mesh=mesh,
        scratch_types=scratch,
    )
    def sc_kernel(t0_hbm, t1_hbm, t0s_hbm, t1s_hbm, grid_hbm,
                  out_hbm, tab8_hbm, *bufs):
        cid = lax.axis_index("c")
        sid = lax.axis_index("s")
        wid = sid * 2 + cid
        cbase = cid * HW

        names = ("g", "idx", "wx", "wy", "rows", "o0", "o1",
                 "gsem", "rsem", "osem")
        B = [dict(zip(names, bufs[:10])), dict(zip(names, bufs[10:20]))]
        pa, pb, pcv, pd, inter0, inter1, psem, qsem = bufs[20:]

        lane = lax.iota(jnp.int32, 16)
        himask = jnp.full((16,), -65536, jnp.int32)
        cols = [jnp.full((16,), t, jnp.int32) for t in range(4)]

        def build_block(k, ib):
            c = jnp.minimum(sid + 16 * k, nblk - 1)
            q0 = c * _PBLK
            copies = [
                pltpu.make_async_copy(t0_hbm.at[pl.ds(q0, _PBLK)], pa, psem),
                pltpu.make_async_copy(t1_hbm.at[pl.ds(q0, _PBLK)], pb, psem),
                pltpu.make_async_copy(t0s_hbm.at[pl.ds(q0, _PBLK)], pcv, psem),
                pltpu.make_async_copy(t1s_hbm.at[pl.ds(q0, _PBLK)], pd, psem),
            ]
            for cp_ in copies:
                cp_.start()
            for cp_ in copies:
                cp_.wait()

            @pl.loop(0, _PBLK, step=16)
            def _(i):
                s = pl.ds(i, 16)
                p = i + lane
                plsc.store_scatter(ib, [p, cols[0]], pa[s])
                plsc.store_scatter(ib, [p, cols[1]], pb[s])
                plsc.store_scatter(ib, [p, cols[2]], pcv[s])
                plsc.store_scatter(ib, [p, cols[3]], pd[s])

            pltpu.async_copy(ib, tab8_hbm.at[pl.ds(cbase + q0, _PBLK)], qsem)

        def wait_block(ib):
            pltpu.make_async_copy(
                ib, tab8_hbm.at[pl.ds(0, _PBLK)], qsem).wait()

        @pl.loop(0, blk_slots // 2)
        def _(m):
            @pl.when(m >= 1)
            def _():
                wait_block(inter0)

            build_block(2 * m, inter0)

            @pl.when(m >= 1)
            def _():
                wait_block(inter1)

            build_block(2 * m + 1, inter1)

        wait_block(inter0)
        wait_block(inter1)
        plsc.subcore_barrier()

        def p_off(t):
            n = wid * imgs_per_worker + t // chunks_per_img
            j = lax.rem(t, chunks_per_img)
            return n * HOWO + j * _CHUNK

        def o_off(t):
            n = wid * imgs_per_worker + t // chunks_per_img
            j = lax.rem(t, chunks_per_img)
            return n * 2 * HOWO + j * _CHUNK

        def start_grid(t, b):
            pltpu.async_copy(grid_hbm.at[pl.ds(2 * p_off(t), 2 * _CHUNK)],
                             B[b]["g"], B[b]["gsem"])

        def wait_grid(b):
            pltpu.make_async_copy(grid_hbm.at[pl.ds(0, 2 * _CHUNK)],
                                  B[b]["g"], B[b]["gsem"]).wait()

        def compute(b):
            g_v, idx_v = B[b]["g"], B[b]["idx"]
            wx_v, wy_v = B[b]["wx"], B[b]["wy"]

            @pl.loop(0, _CHUNK, step=32)
            def _(i):
                for u in (0, 16):
                    iu = i + u
                    base = (iu // 128) * 256 + lax.rem(iu, 128)
                    gx = g_v[pl.ds(base, 16)]
                    gy = g_v[pl.ds(base + 128, 16)]
                    x = jnp.minimum(gx * sx + sx, W - 1.0)
                    y = jnp.minimum(gy * sy + sy, H - 1.0)
                    xi = x.astype(jnp.int32)
                    yi = y.astype(jnp.int32)
                    idx_v[pl.ds(iu, 16)] = xi * H + yi + cbase
                    wx_v[pl.ds(iu, 16)] = x - xi.astype(jnp.float32)
                    wy_v[pl.ds(iu, 16)] = y - yi.astype(jnp.float32)

        def gather_copies(b):
            d = B[b]
            for k in range(_CHUNK // _GSUB):
                s = pl.ds(k * _GSUB, _GSUB)
                yield pltpu.make_async_copy(
                    tab8_hbm.at[d["idx"].at[s]], d["rows"].at[s], d["rsem"])

        def fire(b):
            for c in gather_copies(b):
                c.start()

        def drain(b):
            for c in gather_copies(b):
                c.wait()

        def combine(b):
            d = B[b]
            rows_v = d["rows"]
            wx_v, wy_v, out0_v, out1_v = d["wx"], d["wy"], d["o0"], d["o1"]

            @pl.loop(0, _CHUNK, step=32)
            def _(i):
                for u in (0, 16):
                    s = pl.ds(i + u, 16)
                    p = i + u + lane
                    wx = wx_v[s]
                    wy = wy_v[s]
                    wl0 = plsc.load_gather(rows_v, [p, cols[0]])
                    wl1 = plsc.load_gather(rows_v, [p, cols[1]])
                    wr0 = plsc.load_gather(rows_v, [p, cols[2]])
                    wr1 = plsc.load_gather(rows_v, [p, cols[3]])
                    v00 = plsc.bitcast(wl0 << 16, jnp.float32)
                    v10 = plsc.bitcast(wl0 & himask, jnp.float32)
                    v01 = plsc.bitcast(wr0 << 16, jnp.float32)
                    v11 = plsc.bitcast(wr0 & himask, jnp.float32)
                    u00 = plsc.bitcast(wl1 << 16, jnp.float32)
                    u10 = plsc.bitcast(wl1 & himask, jnp.float32)
                    u01 = plsc.bitcast(wr1 << 16, jnp.float32)
                    u11 = plsc.bitcast(wr1 & himask, jnp.float32)
                    cl0 = v00 + wy * (v10 - v00)
                    cr0 = v01 + wy * (v11 - v01)
                    cl1 = u00 + wy * (u10 - u00)
                    cr1 = u01 + wy * (u11 - u01)
                    out0_v[s] = cl0 + wx * (cr0 - cl0)
                    out1_v[s] = cl1 + wx * (cr1 - cl1)

        def out_copies(t, b):
            d = B[b]
            o0 = o_off(t)
            yield pltpu.make_async_copy(
                d["o0"], out_hbm.at[pl.ds(o0, _CHUNK)], d["osem"])
            yield pltpu.make_async_copy(
                d["o1"], out_hbm.at[pl.ds(o0 + HOWO, _CHUNK)], d["osem"])

        def drain_out(b):
            for c in out_copies(0, b):
                c.wait()

        def stage_a(t, b):
            wait_grid(b)
            compute(b)
            fire(b)

        def stage_b(t, b):
            drain(b)

            @pl.when(t >= 2)
            def _():
                drain_out(b)

            combine(b)
            for c in out_copies(t, b):
                c.start()

        start_grid(0, 0)
        stage_a(0, 0)
        start_grid(1, 1)

        @pl.loop(0, nch, step=2)
        def _(t):
            stage_a(t + 1, 1)

            @pl.when(t + 2 < nch)
            def _():
                start_grid(t + 2, 0)

            stage_b(t, 0)

            @pl.when(t + 2 < nch)
            def _():
                stage_a(t + 2, 0)
                start_grid(t + 3, 1)

            stage_b(t + 1, 1)

        drain_out(0)
        drain_out(1)

    return sc_kernel(tab0, tab1, tab0s, tab1s, grid_lin)[0]


def _pack_vert_pairs(plane_cm):
    lo = lax.bitcast_convert_type(
        plane_cm.astype(jnp.bfloat16), jnp.uint16).astype(jnp.uint32)
    hi = jnp.concatenate([lo[1:], lo[-1:]])
    return (lo | (hi << 16)).astype(jnp.int32)


def kernel(grid, inp):
    N, Ho, Wo, _ = grid.shape
    _, C, H, W = inp.shape
    assert C == 2 and N % _NUM_WORKERS == 0 and (Ho * Wo) % _CHUNK == 0
    assert Wo >= 128 and Wo % 128 == 0 and _CHUNK % 128 == 0
    assert (H * W) % _PBLK == 0 and _PBLK % 16 == 0

    tab0 = _pack_vert_pairs(inp[0, 0].T.reshape(-1))
    tab1 = _pack_vert_pairs(inp[0, 1].T.reshape(-1))
    tab0s = jnp.concatenate([tab0[H:], tab0[-H:]])
    tab1s = jnp.concatenate([tab1[H:], tab1[-H:]])
    grid_lin = grid.transpose(0, 1, 3, 2).reshape(-1)

    out_flat = _sc_grid_sample(tab0, tab1, tab0s, tab1s, grid_lin,
                               H=H, W=W, N=N, Ho=Ho, Wo=Wo)
    return out_flat.reshape(N, C, Ho, Wo)
